# baseline (speedup 1.0000x reference)
import jax
import jax.numpy as jnp
import numpy as np
from jax.experimental import pallas as pl
from jax.experimental.pallas import tpu as pltpu

_NCODONS = 2
_N_STATES = 24
_N_PARAMS = 10


def _static_structure(nCodons=_NCODONS):
    offset = 8 + 3 * nCodons
    idx = [[0, 0], [0, 1], [1, 2], [2, 3]]
    idx += [[3 + i * 3, 4 + i * 3] for i in range(nCodons)]
    idx += [[4 + i * 3, 5 + i * 3] for i in range(nCodons)]
    idx += [[5 + i * 3, 6 + i * 3] for i in range(nCodons)]
    idx += [[3 + i * 3, offset + i * 3] for i in range(nCodons + 1)]
    idx += [[3 + nCodons * 3, 4 + nCodons * 3]]
    idx += [[offset + i * 3, offset + 1 + i * 3] for i in range(nCodons + 1)]
    idx += [[offset + 1 + i * 3, offset + 2 + i * 3] for i in range(nCodons + 1)]
    idx += [[offset + 2 + i * 3, 4 + i * 3] for i in range(nCodons + 1)]
    idx += [[offset + 2 + i * 3, offset + i * 3] for i in range(nCodons + 1)]
    i_del = [3 + i * 3 for i in range(nCodons) for j in range(nCodons - i)]
    j_del = [4 + j * 3 for i in range(1, nCodons + 1) for j in range(i, nCodons + 1)]
    idx += [[i, j] for i, j in zip(i_del, j_del)]
    idx += [[4 + nCodons * 3, 5 + nCodons * 3]]
    idx += [[5 + nCodons * 3, 6 + nCodons * 3]]
    idx += [[6 + nCodons * 3, 7 + nCodons * 3]]
    idx += [[7 + nCodons * 3, 7 + nCodons * 3]]
    idx += [[7 + nCodons * 3, 8 + nCodons * 3 + (nCodons + 1) * 3]]
    idx += [[8 + nCodons * 3 + (nCodons + 1) * 3,
             8 + nCodons * 3 + (nCodons + 1) * 3]]
    idx = np.array(idx, dtype=np.int32)

    nc = nCodons
    a, b, p, e = [], [], [], []

    def add(ai, bi, pi, ei):
        a.append(ai); b.append(bi); p.append(pi); e.append(ei)

    add(1.0, -1.0, 0, 1)
    add(0.0, 1.0, 0, 1)
    for _ in range(2):
        add(1.0, 0.0, 0, 1)
    k = 1
    for i in range(nc):
        add(0.0, 1.0, k + i, 1)
    k += nc
    for _ in range(2 * nc):
        add(1.0, 0.0, 0, 1)
    for i in range(nc + 1):
        add(0.0, 1.0, k + i, 1)
    k += nc + 1
    add(1.0, -1.0, k - 1, 1)
    for _ in range(2 * (nc + 1)):
        add(1.0, 0.0, 0, 1)
    for i in range(nc + 1):
        add(0.0, 1.0, k + i, 1)
    for i in range(nc + 1):
        add(1.0, -1.0, k + i, 1)
    k += nc + 1
    for i, j in zip(i_del, j_del):
        add(1.0, -1.0, k, 1 + int((j - i) / 3))
    k += 1
    for _ in range(6):
        add(1.0, 0.0, 0, 1)

    assert len(a) == len(idx)
    return (idx, np.asarray(a, np.float32), np.asarray(b, np.float32),
            np.asarray(p, np.int32), np.asarray(e, np.int32))


_IDX, _COEF_A, _COEF_B, _PAR, _EXP = _static_structure()
_NK = len(_IDX)


_W = 384
_NPH = 3

_LIVE_TILES = [(q, p) for q in range(_NPH) for p in range(_NPH)
               if not (q == 0 and p == 2) and not (q == 2 and p == 0)]


def _body(w_ref, alpha_ref, out_ref, t_ref):
    @pl.when(pl.program_id(0) == 0)
    def _build_t():
        ws = [w_ref[0, i] for i in range(_N_PARAMS)]
        ri = jax.lax.broadcasted_iota(jnp.int32, (_N_STATES, _N_STATES), 0)
        ci = jax.lax.broadcasted_iota(jnp.int32, (_N_STATES, _N_STATES), 1)
        logits = jnp.zeros((_N_STATES, _N_STATES), jnp.float32)
        maskf = jnp.zeros((_N_STATES, _N_STATES), jnp.float32)
        for t in range(_NK):
            wp = ws[int(_PAR[t])]
            v = wp
            for _ in range(int(_EXP[t]) - 1):
                v = v * wp
            val = float(_COEF_A[t]) + float(_COEF_B[t]) * v
            hot = ((ri == int(_IDX[t, 0])) & (ci == int(_IDX[t, 1])))
            hotf = hot.astype(jnp.float32)
            logits = logits + val * hotf
            maskf = maskf + hotf
        emat = jnp.exp(logits) * maskf
        inv = 1.0 / jnp.sum(emat, axis=1, keepdims=True)
        a_mat = emat * inv

        iu = jax.lax.broadcasted_iota(jnp.int32, (_W, _N_STATES), 0)
        ju = jax.lax.broadcasted_iota(jnp.int32, (_W, _N_STATES), 1)
        u = (iu % _N_STATES == ju).astype(jnp.float32)
        jt = jax.lax.broadcasted_iota(jnp.int32, (_N_STATES, _W), 1)
        rt = jax.lax.broadcasted_iota(jnp.int32, (_N_STATES, _W), 0)
        ut = (jt % _N_STATES == rt).astype(jnp.float32)
        a_ut = jnp.dot(a_mat, ut, preferred_element_type=jnp.float32)
        tiled = jnp.dot(u, a_ut, preferred_element_type=jnp.float32)
        bi = jax.lax.broadcasted_iota(jnp.int32, (_W, _W), 0) // _N_STATES
        bj = jax.lax.broadcasted_iota(jnp.int32, (_W, _W), 1) // _N_STATES
        t_ref[...] = jnp.where(bi == bj, tiled, 0.0)

    xs = [alpha_ref[:, q, :] for q in range(_NPH)]
    ys = [None, None, None]
    for q, p in _LIVE_TILES:
        tqp = t_ref[q * 128:(q + 1) * 128, p * 128:(p + 1) * 128]
        contrib = jnp.dot(xs[q], tqp, preferred_element_type=jnp.float32)
        ys[p] = contrib if ys[p] is None else ys[p] + contrib
    out_ref[...] = jnp.stack(ys, axis=1)


@jax.jit
def kernel(alpha, transition_kernel):
    n = alpha.shape[0]
    groups = n * _N_STATES // _W
    alpha_v = alpha.reshape(groups, _NPH, 128)
    blk = 512
    grid = groups // blk
    w2 = transition_kernel.reshape(1, _N_PARAMS)
    out_v = pl.pallas_call(
        _body,
        grid=(grid,),
        in_specs=[
            pl.BlockSpec(memory_space=pltpu.SMEM),
            pl.BlockSpec((blk, _NPH, 128), lambda i: (i, 0, 0)),
        ],
        out_specs=pl.BlockSpec((blk, _NPH, 128), lambda i: (i, 0, 0)),
        out_shape=jax.ShapeDtypeStruct((groups, _NPH, 128), jnp.float32),
        scratch_shapes=[pltpu.VMEM((_W, _W), jnp.float32)],
        compiler_params=pltpu.CompilerParams(
            dimension_semantics=("arbitrary",),
        ),
    )(w2, alpha_v)
    return out_v.reshape(n, _N_STATES)


# speedup vs baseline: 1.0162x; 1.0162x over previous
import jax
import jax.numpy as jnp
import numpy as np
from jax.experimental import pallas as pl
from jax.experimental.pallas import tpu as pltpu

_NCODONS = 2
_N_STATES = 24
_N_PARAMS = 10


def _static_structure(nCodons=_NCODONS):
    offset = 8 + 3 * nCodons
    idx = [[0, 0], [0, 1], [1, 2], [2, 3]]
    idx += [[3 + i * 3, 4 + i * 3] for i in range(nCodons)]
    idx += [[4 + i * 3, 5 + i * 3] for i in range(nCodons)]
    idx += [[5 + i * 3, 6 + i * 3] for i in range(nCodons)]
    idx += [[3 + i * 3, offset + i * 3] for i in range(nCodons + 1)]
    idx += [[3 + nCodons * 3, 4 + nCodons * 3]]
    idx += [[offset + i * 3, offset + 1 + i * 3] for i in range(nCodons + 1)]
    idx += [[offset + 1 + i * 3, offset + 2 + i * 3] for i in range(nCodons + 1)]
    idx += [[offset + 2 + i * 3, 4 + i * 3] for i in range(nCodons + 1)]
    idx += [[offset + 2 + i * 3, offset + i * 3] for i in range(nCodons + 1)]
    i_del = [3 + i * 3 for i in range(nCodons) for j in range(nCodons - i)]
    j_del = [4 + j * 3 for i in range(1, nCodons + 1) for j in range(i, nCodons + 1)]
    idx += [[i, j] for i, j in zip(i_del, j_del)]
    idx += [[4 + nCodons * 3, 5 + nCodons * 3]]
    idx += [[5 + nCodons * 3, 6 + nCodons * 3]]
    idx += [[6 + nCodons * 3, 7 + nCodons * 3]]
    idx += [[7 + nCodons * 3, 7 + nCodons * 3]]
    idx += [[7 + nCodons * 3, 8 + nCodons * 3 + (nCodons + 1) * 3]]
    idx += [[8 + nCodons * 3 + (nCodons + 1) * 3,
             8 + nCodons * 3 + (nCodons + 1) * 3]]
    idx = np.array(idx, dtype=np.int32)

    nc = nCodons
    a, b, p, e = [], [], [], []

    def add(ai, bi, pi, ei):
        a.append(ai); b.append(bi); p.append(pi); e.append(ei)

    add(1.0, -1.0, 0, 1)
    add(0.0, 1.0, 0, 1)
    for _ in range(2):
        add(1.0, 0.0, 0, 1)
    k = 1
    for i in range(nc):
        add(0.0, 1.0, k + i, 1)
    k += nc
    for _ in range(2 * nc):
        add(1.0, 0.0, 0, 1)
    for i in range(nc + 1):
        add(0.0, 1.0, k + i, 1)
    k += nc + 1
    add(1.0, -1.0, k - 1, 1)
    for _ in range(2 * (nc + 1)):
        add(1.0, 0.0, 0, 1)
    for i in range(nc + 1):
        add(0.0, 1.0, k + i, 1)
    for i in range(nc + 1):
        add(1.0, -1.0, k + i, 1)
    k += nc + 1
    for i, j in zip(i_del, j_del):
        add(1.0, -1.0, k, 1 + int((j - i) / 3))
    k += 1
    for _ in range(6):
        add(1.0, 0.0, 0, 1)

    assert len(a) == len(idx)
    return (idx, np.asarray(a, np.float32), np.asarray(b, np.float32),
            np.asarray(p, np.int32), np.asarray(e, np.int32))


_IDX, _COEF_A, _COEF_B, _PAR, _EXP = _static_structure()
_NK = len(_IDX)


_W = 384
_NPH = 3

_LIVE_TILES = [(q, p) for q in range(_NPH) for p in range(_NPH)
               if not (q == 0 and p == 2) and not (q == 2 and p == 0)]


def _body(w_ref, alpha_ref, out_ref, t_ref):
    @pl.when(pl.program_id(0) == 0)
    def _build_t():
        ws = [w_ref[0, i] for i in range(_N_PARAMS)]
        ri = jax.lax.broadcasted_iota(jnp.int32, (_N_STATES, _N_STATES), 0)
        ci = jax.lax.broadcasted_iota(jnp.int32, (_N_STATES, _N_STATES), 1)
        logits = jnp.zeros((_N_STATES, _N_STATES), jnp.float32)
        maskf = jnp.zeros((_N_STATES, _N_STATES), jnp.float32)
        for t in range(_NK):
            wp = ws[int(_PAR[t])]
            v = wp
            for _ in range(int(_EXP[t]) - 1):
                v = v * wp
            val = float(_COEF_A[t]) + float(_COEF_B[t]) * v
            hot = ((ri == int(_IDX[t, 0])) & (ci == int(_IDX[t, 1])))
            hotf = hot.astype(jnp.float32)
            logits = logits + val * hotf
            maskf = maskf + hotf
        emat = jnp.exp(logits) * maskf
        inv = 1.0 / jnp.sum(emat, axis=1, keepdims=True)
        a_mat = emat * inv

        iu = jax.lax.broadcasted_iota(jnp.int32, (_W, _N_STATES), 0)
        ju = jax.lax.broadcasted_iota(jnp.int32, (_W, _N_STATES), 1)
        u = (iu % _N_STATES == ju).astype(jnp.float32)
        jt = jax.lax.broadcasted_iota(jnp.int32, (_N_STATES, _W), 1)
        rt = jax.lax.broadcasted_iota(jnp.int32, (_N_STATES, _W), 0)
        ut = (jt % _N_STATES == rt).astype(jnp.float32)
        a_ut = jnp.dot(a_mat, ut, preferred_element_type=jnp.float32)
        tiled = jnp.dot(u, a_ut, preferred_element_type=jnp.float32)
        bi = jax.lax.broadcasted_iota(jnp.int32, (_W, _W), 0) // _N_STATES
        bj = jax.lax.broadcasted_iota(jnp.int32, (_W, _W), 1) // _N_STATES
        t_ref[...] = jnp.where(bi == bj, tiled, 0.0)

    x = alpha_ref[...]
    b3 = x.shape[0]
    x3 = x.reshape(b3 // _NPH, _NPH, 128)
    xs = [x3[:, q, :] for q in range(_NPH)]
    ys = [None, None, None]
    for q, p in _LIVE_TILES:
        tqp = t_ref[q * 128:(q + 1) * 128, p * 128:(p + 1) * 128]
        contrib = jnp.dot(xs[q], tqp, preferred_element_type=jnp.float32)
        ys[p] = contrib if ys[p] is None else ys[p] + contrib
    y = jnp.stack(ys, axis=1)
    out_ref[...] = y.reshape(b3, 128)


@jax.jit
def kernel(alpha, transition_kernel):
    n = alpha.shape[0]
    lrows = n * _N_STATES // 128
    alpha_v = alpha.reshape(lrows, 128)
    blk = 1536
    grid = lrows // blk
    w2 = transition_kernel.reshape(1, _N_PARAMS)
    out_v = pl.pallas_call(
        _body,
        grid=(grid,),
        in_specs=[
            pl.BlockSpec(memory_space=pltpu.SMEM),
            pl.BlockSpec((blk, 128), lambda i: (i, 0)),
        ],
        out_specs=pl.BlockSpec((blk, 128), lambda i: (i, 0)),
        out_shape=jax.ShapeDtypeStruct((lrows, 128), jnp.float32),
        scratch_shapes=[pltpu.VMEM((_W, _W), jnp.float32)],
        compiler_params=pltpu.CompilerParams(
            dimension_semantics=("arbitrary",),
        ),
    )(w2, alpha_v)
    return out_v.reshape(n, _N_STATES)


# speedup vs baseline: 1.8875x; 1.8574x over previous
import jax
import jax.numpy as jnp
import numpy as np
from jax.experimental import pallas as pl
from jax.experimental.pallas import tpu as pltpu

_NCODONS = 2
_N_STATES = 24
_N_PARAMS = 10


def _static_structure(nCodons=_NCODONS):
    offset = 8 + 3 * nCodons
    idx = [[0, 0], [0, 1], [1, 2], [2, 3]]
    idx += [[3 + i * 3, 4 + i * 3] for i in range(nCodons)]
    idx += [[4 + i * 3, 5 + i * 3] for i in range(nCodons)]
    idx += [[5 + i * 3, 6 + i * 3] for i in range(nCodons)]
    idx += [[3 + i * 3, offset + i * 3] for i in range(nCodons + 1)]
    idx += [[3 + nCodons * 3, 4 + nCodons * 3]]
    idx += [[offset + i * 3, offset + 1 + i * 3] for i in range(nCodons + 1)]
    idx += [[offset + 1 + i * 3, offset + 2 + i * 3] for i in range(nCodons + 1)]
    idx += [[offset + 2 + i * 3, 4 + i * 3] for i in range(nCodons + 1)]
    idx += [[offset + 2 + i * 3, offset + i * 3] for i in range(nCodons + 1)]
    i_del = [3 + i * 3 for i in range(nCodons) for j in range(nCodons - i)]
    j_del = [4 + j * 3 for i in range(1, nCodons + 1) for j in range(i, nCodons + 1)]
    idx += [[i, j] for i, j in zip(i_del, j_del)]
    idx += [[4 + nCodons * 3, 5 + nCodons * 3]]
    idx += [[5 + nCodons * 3, 6 + nCodons * 3]]
    idx += [[6 + nCodons * 3, 7 + nCodons * 3]]
    idx += [[7 + nCodons * 3, 7 + nCodons * 3]]
    idx += [[7 + nCodons * 3, 8 + nCodons * 3 + (nCodons + 1) * 3]]
    idx += [[8 + nCodons * 3 + (nCodons + 1) * 3,
             8 + nCodons * 3 + (nCodons + 1) * 3]]
    idx = np.array(idx, dtype=np.int32)

    nc = nCodons
    a, b, p, e = [], [], [], []

    def add(ai, bi, pi, ei):
        a.append(ai); b.append(bi); p.append(pi); e.append(ei)

    add(1.0, -1.0, 0, 1)
    add(0.0, 1.0, 0, 1)
    for _ in range(2):
        add(1.0, 0.0, 0, 1)
    k = 1
    for i in range(nc):
        add(0.0, 1.0, k + i, 1)
    k += nc
    for _ in range(2 * nc):
        add(1.0, 0.0, 0, 1)
    for i in range(nc + 1):
        add(0.0, 1.0, k + i, 1)
    k += nc + 1
    add(1.0, -1.0, k - 1, 1)
    for _ in range(2 * (nc + 1)):
        add(1.0, 0.0, 0, 1)
    for i in range(nc + 1):
        add(0.0, 1.0, k + i, 1)
    for i in range(nc + 1):
        add(1.0, -1.0, k + i, 1)
    k += nc + 1
    for i, j in zip(i_del, j_del):
        add(1.0, -1.0, k, 1 + int((j - i) / 3))
    k += 1
    for _ in range(6):
        add(1.0, 0.0, 0, 1)

    assert len(a) == len(idx)
    return (idx, np.asarray(a, np.float32), np.asarray(b, np.float32),
            np.asarray(p, np.int32), np.asarray(e, np.int32))


_IDX, _COEF_A, _COEF_B, _PAR, _EXP = _static_structure()
_NK = len(_IDX)


_W = 384
_NPH = 3

_LIVE_TILES = [(q, p) for q in range(_NPH) for p in range(_NPH)
               if not (q == 0 and p == 2) and not (q == 2 and p == 0)]


def _amat(w_ref):
        ws = [w_ref[0, i] for i in range(_N_PARAMS)]
        ri = jax.lax.broadcasted_iota(jnp.int32, (_N_STATES, _N_STATES), 0)
        ci = jax.lax.broadcasted_iota(jnp.int32, (_N_STATES, _N_STATES), 1)
        logits = jnp.zeros((_N_STATES, _N_STATES), jnp.float32)
        maskf = jnp.zeros((_N_STATES, _N_STATES), jnp.float32)
        for t in range(_NK):
            wp = ws[int(_PAR[t])]
            v = wp
            for _ in range(int(_EXP[t]) - 1):
                v = v * wp
            val = float(_COEF_A[t]) + float(_COEF_B[t]) * v
            hot = ((ri == int(_IDX[t, 0])) & (ci == int(_IDX[t, 1])))
            hotf = hot.astype(jnp.float32)
            logits = logits + val * hotf
            maskf = maskf + hotf
        emat = jnp.exp(logits) * maskf
        inv = 1.0 / jnp.sum(emat, axis=1, keepdims=True)
        return emat * inv


def _body(w_ref, alpha_ref, out_ref):
    a_mat = _amat(w_ref)
    out_ref[...] = jnp.dot(alpha_ref[...], a_mat,
                           preferred_element_type=jnp.float32)


@jax.jit
def kernel(alpha, transition_kernel):
    n = alpha.shape[0]
    blk = 8192
    grid = n // blk
    w2 = transition_kernel.reshape(1, _N_PARAMS)
    return pl.pallas_call(
        _body,
        grid=(grid,),
        in_specs=[
            pl.BlockSpec(memory_space=pltpu.SMEM),
            pl.BlockSpec((blk, _N_STATES), lambda i: (i, 0)),
        ],
        out_specs=pl.BlockSpec((blk, _N_STATES), lambda i: (i, 0)),
        out_shape=jax.ShapeDtypeStruct((n, _N_STATES), jnp.float32),
        compiler_params=pltpu.CompilerParams(
            dimension_semantics=("parallel",),
        ),
    )(w2, alpha)
